# baseline (speedup 1.0000x reference)
import functools

import jax
import jax.numpy as jnp
from jax import lax
from jax.experimental import pallas as pl
from jax.experimental.pallas import tpu as pltpu
from jax.experimental.pallas import tpu_sc as plsc

NUM_EMB = 1000000
DIM = 64
BATCH = 16384


@functools.cache
def _build():
    info = plsc.get_sparse_core_info()
    NC, NS = info.num_cores, info.num_subcores
    NW = NC * NS
    bpw = BATCH // NW

    mesh = plsc.VectorSubcoreMesh(core_axis_name="c", subcore_axis_name="s")

    @functools.partial(
        pl.kernel,
        mesh=mesh,
        compiler_params=pltpu.CompilerParams(needs_layout_passes=False),
        out_type=jax.ShapeDtypeStruct((BATCH, DIM), jnp.float32),
        scratch_types=[
            pltpu.VMEM((bpw,), jnp.int32),
            pltpu.VMEM((bpw,), jnp.int32),
            pltpu.VMEM((bpw // 2, 2 * DIM), jnp.float32),
            pltpu.VMEM((bpw, DIM), jnp.float32),
            pltpu.SemaphoreType.DMA,
        ],
    )
    def gather_rows(t2, idx_hbm, out, idx_v, row_v, pair_v, out_v, sem):
        wid = lax.axis_index("s") * NC + lax.axis_index("c")
        base = wid * bpw
        pltpu.sync_copy(idx_hbm.at[pl.ds(base, bpw)], idx_v)

        def shift(ch, carry):
            c = idx_v[pl.ds(ch * 16, 16)]
            row_v[pl.ds(ch * 16, 16)] = c >> 1
            return carry

        lax.fori_loop(0, bpw // 16, shift, 0)

        lanes = lax.iota(jnp.int32, 16)
        half_rows = bpw // 2

        for rnd in range(2):
            for part in range(half_rows // 128):
                pltpu.async_copy(
                    t2.at[row_v.at[pl.ds(rnd * half_rows + part * 128, 128)]],
                    pair_v.at[pl.ds(part * 128, 128), :],
                    sem,
                )
            for part in range(half_rows // 128):
                pltpu.make_async_copy(
                    t2.at[pl.ds(0, 128)], pair_v.at[pl.ds(0, 128), :], sem
                ).wait()

            def extract(m, carry, rnd=rnd):
                half = (idx_v[pl.ds(rnd * half_rows + m * 16, 16)] & 1) * DIM
                rows = m * 16 + lanes
                out_rows = rnd * half_rows + rows
                for j in range(DIM):
                    x = plsc.load_gather(pair_v, [rows, half + j])
                    plsc.store_scatter(
                        out_v, [out_rows, jnp.full((16,), j, jnp.int32)], x
                    )
                return carry

            lax.fori_loop(0, half_rows // 16, extract, 0)

        pltpu.sync_copy(out_v, out.at[pl.ds(base, bpw), :])

    return gather_rows


def kernel(table, batch_idx):
    f = _build()
    t2 = table.reshape(NUM_EMB // 2, 2 * DIM)
    return f(t2, batch_idx.astype(jnp.int32))


# speedup vs baseline: 3.0966x; 3.0966x over previous
import functools

import jax
import jax.numpy as jnp
from jax import lax
from jax.experimental import pallas as pl
from jax.experimental.pallas import tpu as pltpu
from jax.experimental.pallas import tpu_sc as plsc

NUM_EMB = 1000000
DIM = 64
BATCH = 16384


@functools.cache
def _build():
    info = plsc.get_sparse_core_info()
    NC, NS = info.num_cores, info.num_subcores
    NW = NC * NS
    bpw = BATCH // NW
    NBUF = 4

    mesh = plsc.VectorSubcoreMesh(core_axis_name="c", subcore_axis_name="s")

    @functools.partial(
        pl.kernel,
        mesh=mesh,
        compiler_params=pltpu.CompilerParams(needs_layout_passes=False),
        out_type=jax.ShapeDtypeStruct((BATCH, DIM), jnp.float32),
        scratch_types=[
            pltpu.VMEM((bpw + 16,), jnp.int32),
            pltpu.VMEM((NBUF, DIM, 128), jnp.float32),
            pltpu.VMEM((bpw, DIM), jnp.float32),
            pltpu.SemaphoreType.DMA,
        ],
    )
    def gather_cols(tt, idx_hbm, out, idx_v, rbuf, out_v, sem):
        wid = lax.axis_index("s") * NC + lax.axis_index("c")
        base = wid * bpw
        pltpu.sync_copy(idx_hbm.at[pl.ds(base, bpw)], idx_v.at[pl.ds(0, bpw)])

        lanes = lax.iota(jnp.int32, 16)

        def fetch(k, slot):
            tc = idx_v[pl.ds(k, 16)][0] >> 7
            pltpu.async_copy(
                tt.at[:, pl.ds(tc * 128, 128)], rbuf.at[slot], sem
            )

        for k in range(NBUF):
            fetch(k, k)

        def body(k, carry):
            slot = k & (NBUF - 1)
            pltpu.make_async_copy(
                tt.at[:, pl.ds(0, 128)], rbuf.at[0], sem
            ).wait()
            cc = jnp.full((16,), idx_v[pl.ds(k, 16)][0] & 127, jnp.int32)
            kk = jnp.full((16,), k, jnp.int32)
            for g in range(DIM // 16):
                rows = g * 16 + lanes
                x = plsc.load_gather(rbuf.at[slot], [rows, cc])
                plsc.store_scatter(out_v, [kk, rows], x)

            @pl.when(k < bpw - NBUF)
            def _next():
                fetch(k + NBUF, slot)

            return carry

        lax.fori_loop(0, bpw, body, 0)
        pltpu.sync_copy(out_v, out.at[pl.ds(base, bpw), :])

    return gather_cols


def kernel(table, batch_idx):
    f = _build()
    return f(table.T, batch_idx.astype(jnp.int32))
